# Initial kernel scaffold; baseline (speedup 1.0000x reference)
import jax
import jax.numpy as jnp
from jax.experimental import pallas as pl


def kernel(x, pos, edge_index, batch, conv1_W1, conv1_b1, conv1_g1, conv1_bt1, conv1_W2, conv1_b2, conv2_W1, conv2_b1, conv2_g1, conv2_bt1, conv2_W2, conv2_b2, conv3_W1, conv3_b1, conv3_g1, conv3_bt1, conv3_W2, conv3_b2, lin1_W, lin1_b, lin2_W, lin2_b):
    raise NotImplementedError("write your pallas kernel here")


# speedup vs baseline: 3.1582x; 3.1582x over previous
import functools

import jax
import jax.numpy as jnp
from jax import lax
from jax.experimental import pallas as pl
from jax.experimental.pallas import tpu as pltpu
from jax.experimental.pallas import tpu_sc as plsc

N = 50000
E = 800000
G = 2048
H = 64

WIN = 80
NSUB = 16
NCORE = 2

@functools.cache
def _mesh():
    return plsc.VectorSubcoreMesh(
        core_axis_name="c", subcore_axis_name="s",
        num_cores=NCORE, num_subcores=NSUB)


def _zero_fill_2d(ref, nrows, d):
    z16 = jnp.zeros((16,), jnp.float32)

    @pl.loop(0, nrows)
    def _(i):
        for k in range(d // 16):
            ref[i, pl.ds(16 * k, 16)] = z16


def _sc_agg_half(h_split, src, dst):
    d = 32
    n_per_sub = N // NSUB
    nwin_per_sub = (E // WIN) // NSUB

    nrw_full = (N // WIN) // NSUB
    nrw_rem = (N // WIN) - nrw_full * NSUB

    @functools.partial(
        pl.kernel,
        out_type=jax.ShapeDtypeStruct((NCORE, N, d), jnp.float32),
        mesh=_mesh(),
        compiler_params=pltpu.CompilerParams(use_tc_tiling_on_sc=False, needs_layout_passes=False),
        scratch_types=[
            pltpu.VMEM_SHARED((N, d), jnp.float32),
            pltpu.VMEM((WIN, d), jnp.float32),
            pltpu.VMEM((WIN,), jnp.int32),
            pltpu.VMEM((WIN,), jnp.int32),
            pltpu.VMEM((WIN, d), jnp.float32),
            pltpu.SemaphoreType.DMA,
        ],
    )
    def k(h_hbm, src_hbm, dst_hbm, out_hbm, acc, zbuf, sidx, didx, rows, sem):
        c = lax.axis_index("c")
        s = lax.axis_index("s")
        _zero_fill_2d(zbuf, WIN, d)
        nrw = jnp.where(s < nrw_rem, nrw_full + 1, nrw_full)

        @pl.loop(0, nrw)
        def _(i):
            pltpu.sync_copy(zbuf, acc.at[pl.ds((s + NSUB * i) * WIN, WIN)])

        plsc.subcore_barrier()

        @pl.loop(0, nwin_per_sub)
        def _(j):
            base = (s + NSUB * j) * WIN
            pltpu.sync_copy(src_hbm.at[pl.ds(base, WIN)], sidx)
            pltpu.sync_copy(dst_hbm.at[pl.ds(base, WIN)], didx)
            pltpu.async_copy(h_hbm.at[c].at[sidx], rows, sem).wait()
            pltpu.sync_copy(rows, acc.at[didx], add=True)

        plsc.subcore_barrier()

        @pl.loop(0, nrw)
        def _(i):
            base = (s + NSUB * i) * WIN
            pltpu.sync_copy(acc.at[pl.ds(base, WIN)],
                            out_hbm.at[c].at[pl.ds(base, WIN)])

    return k(h_split, src, dst)


def _sc_agg_l1(h0p, src, dst):
    d = 16
    n_per_sub = N // NSUB
    e_half = E // NCORE
    nwin_half = e_half // WIN

    nrw_full = (N // WIN) // NSUB
    nrw_rem = (N // WIN) - nrw_full * NSUB

    @functools.partial(
        pl.kernel,
        out_type=jax.ShapeDtypeStruct((NCORE, N, d), jnp.float32),
        mesh=_mesh(),
        compiler_params=pltpu.CompilerParams(use_tc_tiling_on_sc=False, needs_layout_passes=False),
        scratch_types=[
            pltpu.VMEM_SHARED((N, d), jnp.float32),
            pltpu.VMEM((WIN, d), jnp.float32),
            pltpu.VMEM((WIN,), jnp.int32),
            pltpu.VMEM((WIN,), jnp.int32),
            pltpu.VMEM((WIN, d), jnp.float32),
            pltpu.SemaphoreType.DMA,
        ],
    )
    def k(h_hbm, src_hbm, dst_hbm, out_hbm, acc, zbuf, sidx, didx, rows, sem):
        c = lax.axis_index("c")
        s = lax.axis_index("s")
        _zero_fill_2d(zbuf, WIN, d)
        nrw = jnp.where(s < nrw_rem, nrw_full + 1, nrw_full)

        @pl.loop(0, nrw)
        def _(i):
            pltpu.sync_copy(zbuf, acc.at[pl.ds((s + NSUB * i) * WIN, WIN)])

        plsc.subcore_barrier()

        nw_full = nwin_half // NSUB
        rem = nwin_half - nw_full * NSUB
        nw = jnp.where(s < rem, nw_full + 1, nw_full)

        @pl.loop(0, nw)
        def _(j):
            base = c * e_half + (s + NSUB * j) * WIN
            pltpu.sync_copy(src_hbm.at[pl.ds(base, WIN)], sidx)
            pltpu.sync_copy(dst_hbm.at[pl.ds(base, WIN)], didx)
            pltpu.async_copy(h_hbm.at[sidx], rows, sem).wait()
            pltpu.sync_copy(rows, acc.at[didx], add=True)

        plsc.subcore_barrier()

        @pl.loop(0, nrw)
        def _(i):
            base = (s + NSUB * i) * WIN
            pltpu.sync_copy(acc.at[pl.ds(base, WIN)],
                            out_hbm.at[c].at[pl.ds(base, WIN)])

    return k(h0p, src, dst)


def _sc_pool(h_split, batch):
    d = 32
    nwin_total = N // WIN
    g_per_sub = G // NSUB

    @functools.partial(
        pl.kernel,
        out_type=(
            jax.ShapeDtypeStruct((NCORE, G, d), jnp.float32),
            jax.ShapeDtypeStruct((NCORE, NSUB, G), jnp.float32),
        ),
        mesh=_mesh(),
        compiler_params=pltpu.CompilerParams(use_tc_tiling_on_sc=False, needs_layout_passes=False),
        scratch_types=[
            pltpu.VMEM_SHARED((G, d), jnp.float32),
            pltpu.VMEM((g_per_sub, d), jnp.float32),
            pltpu.VMEM((G,), jnp.float32),
            pltpu.VMEM((WIN,), jnp.int32),
            pltpu.VMEM((WIN, d), jnp.float32),
        ],
    )
    def k(h_hbm, b_hbm, sum_hbm, hist_hbm, acc, zbuf, histl, bidx, rows):
        c = lax.axis_index("c")
        s = lax.axis_index("s")
        _zero_fill_2d(zbuf, g_per_sub, d)
        pltpu.sync_copy(zbuf, acc.at[pl.ds(s * g_per_sub, g_per_sub)])

        @pl.loop(0, G // 16)
        def _(i):
            histl[pl.ds(16 * i, 16)] = jnp.zeros((16,), jnp.float32)

        plsc.subcore_barrier()

        nw_full = nwin_total // NSUB
        rem = nwin_total - nw_full * NSUB
        nw = jnp.where(s < rem, nw_full + 1, nw_full)
        ones16 = jnp.ones((16,), jnp.float32)

        @pl.loop(0, nw)
        def _(j):
            base = (s + NSUB * j) * WIN
            pltpu.sync_copy(h_hbm.at[c].at[pl.ds(base, WIN)], rows)
            pltpu.sync_copy(b_hbm.at[pl.ds(base, WIN)], bidx)
            pltpu.sync_copy(rows, acc.at[bidx], add=True)
            for q in range(WIN // 16):
                iv = bidx[pl.ds(16 * q, 16)]
                plsc.addupdate_scatter(histl, [iv], ones16)

        plsc.subcore_barrier()
        pltpu.sync_copy(
            acc.at[pl.ds(s * g_per_sub, g_per_sub)],
            sum_hbm.at[c].at[pl.ds(s * g_per_sub, g_per_sub)],
        )
        pltpu.sync_copy(histl, hist_hbm.at[c].at[s])

    return k(h_split, batch)


def _tc_dense_a(h_split, agg_split, W1, b1):
    R = 2000
    nb = N // R

    def body(h_ref, a_ref, w_ref, b_ref, z_ref, st_ref, s_acc, q_acc):
        i = pl.program_id(0)
        u = jnp.concatenate(
            [h_ref[0] + a_ref[0], h_ref[1] + a_ref[1]], axis=1)
        z = jnp.dot(u, w_ref[...], preferred_element_type=jnp.float32)
        z = z + b_ref[...]
        z_ref[...] = z

        @pl.when(i == 0)
        def _():
            s_acc[...] = jnp.zeros_like(s_acc)
            q_acc[...] = jnp.zeros_like(q_acc)

        s_acc[...] += jnp.sum(z, axis=0, keepdims=True)
        q_acc[...] += jnp.sum(z * z, axis=0, keepdims=True)

        @pl.when(i == nb - 1)
        def _():
            st_ref[0:1, :] = s_acc[...]
            st_ref[1:2, :] = q_acc[...]

    return pl.pallas_call(
        body,
        grid=(nb,),
        in_specs=[
            pl.BlockSpec((2, R, 32), lambda i: (0, i, 0)),
            pl.BlockSpec((2, R, 32), lambda i: (0, i, 0)),
            pl.BlockSpec((H, H), lambda i: (0, 0)),
            pl.BlockSpec((1, H), lambda i: (0, 0)),
        ],
        out_specs=[
            pl.BlockSpec((R, H), lambda i: (i, 0)),
            pl.BlockSpec((2, H), lambda i: (0, 0)),
        ],
        out_shape=[
            jax.ShapeDtypeStruct((N, H), jnp.float32),
            jax.ShapeDtypeStruct((2, H), jnp.float32),
        ],
        scratch_shapes=[
            pltpu.VMEM((1, H), jnp.float32),
            pltpu.VMEM((1, H), jnp.float32),
        ],
    )(h_split, agg_split, W1, b1)


def _tc_dense_a1(h0p, agg2, W1p, b1):
    R = 2000
    nb = N // R

    def body(h_ref, a_ref, w_ref, b_ref, z_ref, st_ref, s_acc, q_acc):
        i = pl.program_id(0)
        u = h_ref[...] + a_ref[0] + a_ref[1]
        z = jnp.dot(u, w_ref[...], preferred_element_type=jnp.float32)
        z = z + b_ref[...]
        z_ref[...] = z

        @pl.when(i == 0)
        def _():
            s_acc[...] = jnp.zeros_like(s_acc)
            q_acc[...] = jnp.zeros_like(q_acc)

        s_acc[...] += jnp.sum(z, axis=0, keepdims=True)
        q_acc[...] += jnp.sum(z * z, axis=0, keepdims=True)

        @pl.when(i == nb - 1)
        def _():
            st_ref[0:1, :] = s_acc[...]
            st_ref[1:2, :] = q_acc[...]

    return pl.pallas_call(
        body,
        grid=(nb,),
        in_specs=[
            pl.BlockSpec((R, 16), lambda i: (i, 0)),
            pl.BlockSpec((2, R, 16), lambda i: (0, i, 0)),
            pl.BlockSpec((16, H), lambda i: (0, 0)),
            pl.BlockSpec((1, H), lambda i: (0, 0)),
        ],
        out_specs=[
            pl.BlockSpec((R, H), lambda i: (i, 0)),
            pl.BlockSpec((2, H), lambda i: (0, 0)),
        ],
        out_shape=[
            jax.ShapeDtypeStruct((N, H), jnp.float32),
            jax.ShapeDtypeStruct((2, H), jnp.float32),
        ],
        scratch_shapes=[
            pltpu.VMEM((1, H), jnp.float32),
            pltpu.VMEM((1, H), jnp.float32),
        ],
    )(h0p, agg2, W1p, b1)


def _tc_dense_b(z, stats, g1, bt1, W2, b2):
    R = 2000
    nb = N // R
    inv_n = 1.0 / float(N)

    def body(z_ref, st_ref, g_ref, bt_ref, w_ref, b_ref, o_ref):
        mean = st_ref[0:1, :] * inv_n
        var = st_ref[1:2, :] * inv_n - mean * mean
        inv = lax.rsqrt(var + 1e-5)
        zn = (z_ref[...] - mean) * inv * g_ref[...] + bt_ref[...]
        a = jnp.maximum(zn, 0.0)
        hh = jnp.dot(a, w_ref[...], preferred_element_type=jnp.float32)
        hh = jnp.maximum(hh + b_ref[...], 0.0)
        o_ref[0] = hh[:, 0:32]
        o_ref[1] = hh[:, 32:64]

    return pl.pallas_call(
        body,
        grid=(nb,),
        in_specs=[
            pl.BlockSpec((R, H), lambda i: (i, 0)),
            pl.BlockSpec((2, H), lambda i: (0, 0)),
            pl.BlockSpec((1, H), lambda i: (0, 0)),
            pl.BlockSpec((1, H), lambda i: (0, 0)),
            pl.BlockSpec((H, H), lambda i: (0, 0)),
            pl.BlockSpec((1, H), lambda i: (0, 0)),
        ],
        out_specs=pl.BlockSpec((2, R, 32), lambda i: (0, i, 0)),
        out_shape=jax.ShapeDtypeStruct((2, N, 32), jnp.float32),
    )(z, stats, g1, bt1, W2, b2)


def _tc_final(p1, p2, p3, cnt, lin1_W, lin1_b, lin2_W, lin2_b):
    def body(p1_ref, p2_ref, p3_ref, c_ref, w1_ref, b1_ref, w2_ref, b2_ref,
             o_ref):
        cnts = jnp.maximum(jnp.sum(c_ref[...], axis=0), 1.0)[:, None]
        pool = jnp.concatenate(
            [p1_ref[0], p1_ref[1], p2_ref[0], p2_ref[1], p3_ref[0],
             p3_ref[1]], axis=1) / cnts
        y = jnp.dot(pool, w1_ref[...], preferred_element_type=jnp.float32)
        y = jnp.maximum(y + b1_ref[...], 0.0)
        o = jnp.dot(y, w2_ref[...], preferred_element_type=jnp.float32)
        o_ref[...] = o + b2_ref[...]

    return pl.pallas_call(
        body,
        out_shape=jax.ShapeDtypeStruct((G, 1), jnp.float32),
    )(p1, p2, p3, cnt, lin1_W, lin1_b, lin2_W, lin2_b)


def kernel(x, pos, edge_index, batch,
           conv1_W1, conv1_b1, conv1_g1, conv1_bt1, conv1_W2, conv1_b2,
           conv2_W1, conv2_b1, conv2_g1, conv2_bt1, conv2_W2, conv2_b2,
           conv3_W1, conv3_b1, conv3_g1, conv3_bt1, conv3_W2, conv3_b2,
           lin1_W, lin1_b, lin2_W, lin2_b):
    src = edge_index[0]
    dst = edge_index[1]
    h0p = jnp.pad(jnp.concatenate([x, pos], axis=1), ((0, 0), (0, 2)))
    W1p = jnp.pad(conv1_W1, ((0, 2), (0, 0)))

    row = lambda v: v.reshape(1, -1)

    agg0 = _sc_agg_l1(h0p, src, dst)
    z1, s1 = _tc_dense_a1(h0p, agg0, W1p, row(conv1_b1))
    h1s = _tc_dense_b(z1, s1, row(conv1_g1), row(conv1_bt1), conv1_W2,
                      row(conv1_b2))

    a1 = _sc_agg_half(h1s, src, dst)
    z2, s2 = _tc_dense_a(h1s, a1, conv2_W1, row(conv2_b1))
    h2s = _tc_dense_b(z2, s2, row(conv2_g1), row(conv2_bt1), conv2_W2,
                      row(conv2_b2))

    a2 = _sc_agg_half(h2s, src, dst)
    z3, s3 = _tc_dense_a(h2s, a2, conv3_W1, row(conv3_b1))
    h3s = _tc_dense_b(z3, s3, row(conv3_g1), row(conv3_bt1), conv3_W2,
                      row(conv3_b2))

    p1, c1 = _sc_pool(h1s, batch)
    p2, _ = _sc_pool(h2s, batch)
    p3, _ = _sc_pool(h3s, batch)

    return _tc_final(p1, p2, p3, c1[0], lin1_W, row(lin1_b), lin2_W,
                     row(lin2_b))
